# baseline (speedup 1.0000x reference)
import functools
import math

import jax
import jax.numpy as jnp
from jax import lax
from jax.experimental import pallas as pl
from jax.experimental.pallas import tpu as pltpu
from jax.experimental.pallas import tpu_sc as plsc

VOCAB_SIZE = 1000000
D_MODEL = 64
SCALE = math.sqrt(D_MODEL)

NC = 2
NS = 16
NW = NC * NS

B_ROWS = 4096 * 200
BPW = B_ROWS // NW
CHUNK = 128
NCH = BPW // CHUNK
M = 8
L = 4


def _emb_body(idx_hbm, table_hbm, out_hbm, idx_v, buf_v, *sems):
    gsem = sems[:M]
    ssem = sems[M:2 * M]
    sem_i = sems[2 * M]
    wid = lax.axis_index("s") * NC + lax.axis_index("c")
    base = wid * BPW

    pltpu.async_copy(idx_hbm.at[wid], idx_v, sem_i).wait()

    def start_gather(m, c):
        pltpu.make_async_copy(table_hbm.at[idx_v.at[c]], buf_v.at[m],
                              gsem[m]).start()

    def wait_gather(m, c):
        pltpu.make_async_copy(table_hbm.at[idx_v.at[c]], buf_v.at[m],
                              gsem[m]).wait()

    def start_store(m, c):
        pltpu.make_async_copy(buf_v.at[m],
                              out_hbm.at[pl.ds(base + c * CHUNK, CHUNK)],
                              ssem[m]).start()

    def wait_store(m, c):
        pltpu.make_async_copy(buf_v.at[m],
                              out_hbm.at[pl.ds(base + c * CHUNK, CHUNK)],
                              ssem[m]).wait()

    for c0 in range(L):
        start_gather(c0, c0)

    def ring(g, _):
        for m in range(M):
            q = g * M + m
            m_next = (m + L) % M

            @pl.when(q + L < NCH)
            def _():
                @pl.when(q + L >= M)
                def _():
                    wait_store(m_next, q + L - M)

                start_gather(m_next, q + L)

            wait_gather(m, q)
            start_store(m, q)
        return 0

    lax.fori_loop(0, NCH // M, ring, 0)

    for t in range(M):
        c = NCH - M + t
        wait_store(c % M, c)


@jax.jit
def _emb(x_grouped, table_scaled):
    mesh = plsc.VectorSubcoreMesh(
        core_axis_name="c", subcore_axis_name="s", num_cores=NC,
        num_subcores=NS)
    f = functools.partial(
        pl.kernel,
        out_type=jax.ShapeDtypeStruct((B_ROWS, D_MODEL), jnp.float32),
        mesh=mesh,
        scratch_types=[
            pltpu.VMEM((NCH, CHUNK), jnp.int32),
            pltpu.VMEM((M, CHUNK, D_MODEL), jnp.float32),
        ] + [pltpu.SemaphoreType.DMA] * (2 * M + 1),
        compiler_params=pltpu.CompilerParams(use_tc_tiling_on_sc=False),
    )(_emb_body)
    return f(x_grouped, table_scaled)


def kernel(x, table):
    x_grouped = x.astype(jnp.int32).reshape(NW, NCH, CHUNK)
    table_scaled = table * jnp.float32(SCALE)
    out = _emb(x_grouped, table_scaled)
    return out.reshape(x.shape[0], x.shape[1], D_MODEL)


# speedup vs baseline: 1.0130x; 1.0130x over previous
import functools
import math

import jax
import jax.numpy as jnp
from jax import lax
from jax.experimental import pallas as pl
from jax.experimental.pallas import tpu as pltpu
from jax.experimental.pallas import tpu_sc as plsc

VOCAB_SIZE = 1000000
D_MODEL = 64
SCALE = math.sqrt(D_MODEL)

NC = 2
NS = 16
NW = NC * NS
LANES = 16

B_ROWS = 4096 * 200
HALVES = 2
HROWS = B_ROWS // HALVES
BPW = HROWS // NW
CHUNK = 128
NCH = BPW // CHUNK
K = 4
NG = NCH // K


def _emb_body(idx_hbm, table_hbm, out_hbm, idx_v, in_v, out_v, *sems):
    gsem = sems[:K]
    ssem = sems[K:2 * K]
    sem_i = sems[2 * K]
    wid = lax.axis_index("s") * NC + lax.axis_index("c")
    base = wid * BPW

    pltpu.async_copy(idx_hbm.at[wid], idx_v, sem_i).wait()

    def start_gather(b, c):
        pltpu.make_async_copy(table_hbm.at[idx_v.at[c]], in_v.at[b],
                              gsem[b]).start()

    def wait_gather(b, c):
        pltpu.make_async_copy(table_hbm.at[idx_v.at[c]], in_v.at[b],
                              gsem[b]).wait()

    def start_store(b, c):
        pltpu.make_async_copy(out_v.at[b],
                              out_hbm.at[pl.ds(base + c * CHUNK, CHUNK)],
                              ssem[b]).start()

    def wait_store(b, c):
        pltpu.make_async_copy(out_v.at[b],
                              out_hbm.at[pl.ds(base + c * CHUNK, CHUNK)],
                              ssem[b]).wait()

    def scale_rows(b):
        src = in_v.at[b]
        dst = out_v.at[b]

        def row_body(i, _):
            for j in range(D_MODEL // LANES):
                sl = pl.ds(j * LANES, LANES)
                dst[i, sl] = src[i, sl] * SCALE
            return 0

        lax.fori_loop(0, CHUNK, row_body, 0, unroll=8)

    for b in range(K):
        start_gather(b, b)

    def group_body(g, _):
        for b in range(K):
            c = g * K + b
            wait_gather(b, c)

            @pl.when(g > 0)
            def _():
                wait_store(b, c - K)

            scale_rows(b)
            start_store(b, c)
            start_gather(b, c + K)
        return 0

    lax.fori_loop(0, NG - 1, group_body, 0)

    for b in range(K):
        c = (NG - 1) * K + b
        wait_gather(b, c)
        wait_store(b, c - K)
        scale_rows(b)
        start_store(b, c)
    for b in range(K):
        wait_store(b, (NG - 1) * K + b)


@jax.jit
def _emb(x_half, table):
    mesh = plsc.VectorSubcoreMesh(
        core_axis_name="c", subcore_axis_name="s", num_cores=NC,
        num_subcores=NS)
    f = functools.partial(
        pl.kernel,
        out_type=jax.ShapeDtypeStruct((HROWS, D_MODEL), jnp.float32),
        mesh=mesh,
        scratch_types=[
            pltpu.VMEM((NCH, CHUNK), jnp.int32),
            pltpu.VMEM((K, CHUNK, D_MODEL), jnp.float32),
            pltpu.VMEM((K, CHUNK, D_MODEL), jnp.float32),
        ] + [pltpu.SemaphoreType.DMA] * (2 * K + 1),
        compiler_params=pltpu.CompilerParams(use_tc_tiling_on_sc=False),
    )(_emb_body)
    return f(x_half, table)


def kernel(x, table):
    x_grouped = x.astype(jnp.int32).reshape(HALVES, NW, NCH, CHUNK)
    halves = [_emb(x_grouped[h], table) for h in range(HALVES)]
    out = jnp.concatenate(halves, axis=0)
    return out.reshape(x.shape[0], x.shape[1], D_MODEL)


# speedup vs baseline: 1.1431x; 1.1284x over previous
import functools
import math

import jax
import jax.numpy as jnp
from jax import lax
from jax.experimental import pallas as pl
from jax.experimental.pallas import tpu as pltpu
from jax.experimental.pallas import tpu_sc as plsc

VOCAB_SIZE = 1000000
D_MODEL = 64
SCALE = math.sqrt(D_MODEL)

NC = 2
NS = 16
NW = NC * NS
LANES = 16

B_ROWS = 4096 * 200
BPW = B_ROWS // NW
CHUNK = 128
NCH = BPW // CHUNK
K = 4
NG = NCH // K


def _emb_body(idx_hbm, table_hbm, out_hbm, idx_v, in_v, out_v, *sems):
    gsem = sems[:K]
    ssem = sems[K:2 * K]
    sem_i = sems[2 * K]
    wid = lax.axis_index("s") * NC + lax.axis_index("c")
    base = wid * BPW

    pltpu.async_copy(idx_hbm.at[wid], idx_v, sem_i).wait()

    def start_gather(b, c):
        pltpu.make_async_copy(table_hbm.at[idx_v.at[c]], in_v.at[b],
                              gsem[b]).start()

    def wait_gather(b, c):
        pltpu.make_async_copy(table_hbm.at[idx_v.at[c]], in_v.at[b],
                              gsem[b]).wait()

    def start_store(b, c):
        pltpu.make_async_copy(out_v.at[b],
                              out_hbm.at[pl.ds(base + c * CHUNK, CHUNK)],
                              ssem[b]).start()

    def wait_store(b, c):
        pltpu.make_async_copy(out_v.at[b],
                              out_hbm.at[pl.ds(base + c * CHUNK, CHUNK)],
                              ssem[b]).wait()

    def scale_rows(b):
        src = in_v.at[b]
        dst = out_v.at[b]

        def row_body(i, _):
            for j in range(D_MODEL // LANES):
                sl = pl.ds(j * LANES, LANES)
                dst[i, sl] = src[i, sl] * SCALE
            return 0

        lax.fori_loop(0, CHUNK, row_body, 0, unroll=8)

    for b in range(K):
        start_gather(b, b)

    def group_body(g, _):
        for b in range(K):
            c = g * K + b
            wait_gather(b, c)

            @pl.when(g > 0)
            def _():
                wait_store(b, c - K)

            scale_rows(b)
            start_store(b, c)
            start_gather(b, c + K)
        return 0

    lax.fori_loop(0, NG - 1, group_body, 0)

    for b in range(K):
        c = (NG - 1) * K + b
        wait_gather(b, c)
        wait_store(b, c - K)
        scale_rows(b)
        start_store(b, c)
    for b in range(K):
        wait_store(b, (NG - 1) * K + b)


@jax.jit
def _emb(x_grouped, table):
    mesh = plsc.VectorSubcoreMesh(
        core_axis_name="c", subcore_axis_name="s", num_cores=NC,
        num_subcores=NS)
    f = functools.partial(
        pl.kernel,
        out_type=jax.ShapeDtypeStruct((B_ROWS, D_MODEL), jnp.float32),
        mesh=mesh,
        scratch_types=[
            pltpu.VMEM((NCH, CHUNK), jnp.int32),
            pltpu.VMEM((K, CHUNK, D_MODEL), jnp.float32),
            pltpu.VMEM((K, CHUNK, D_MODEL), jnp.float32),
        ] + [pltpu.SemaphoreType.DMA] * (2 * K + 1),
        compiler_params=pltpu.CompilerParams(use_tc_tiling_on_sc=False),
    )(_emb_body)
    return f(x_grouped, table)


def kernel(x, table):
    x_grouped = x.astype(jnp.int32).reshape(NW, NCH, CHUNK)
    out = _emb(x_grouped, table)
    return out.reshape(x.shape[0], x.shape[1], D_MODEL)
